# Initial kernel scaffold; baseline (speedup 1.0000x reference)
import jax
import jax.numpy as jnp
from jax.experimental import pallas as pl


def kernel(z_e, emb):
    raise NotImplementedError("write your pallas kernel here")


# speedup vs baseline: 2.1086x; 2.1086x over previous
import functools

import jax
import jax.numpy as jnp
from jax.experimental import pallas as pl
from jax.experimental.pallas import tpu as pltpu

K = 1024
D = 64
BETA = 0.25
N = 32 * 32 * 32
ROWS = 1024
STEPS = N // ROWS


def _vq_kernel(z_ref, embt_ref, idx_ref, zqt_ref, scal_ref,
               counts_acc, mind_acc):
    b = pl.program_id(0)

    @pl.when(b == 0)
    def _init():
        counts_acc[...] = jnp.zeros_like(counts_acc)
        mind_acc[...] = jnp.zeros_like(mind_acc)

    z = z_ref[0]
    embt = embt_ref[...]

    z2 = jnp.sum(z * z, axis=1, keepdims=True)
    e2 = jnp.sum(embt * embt, axis=0, keepdims=True)
    c = jax.lax.dot_general(z, embt, (((1,), (0,)), ((), ())),
                            preferred_element_type=jnp.float32)
    dist = (z2 + e2) - 2.0 * c

    rowmin = jnp.min(dist, axis=1, keepdims=True)
    kio = jax.lax.broadcasted_iota(jnp.int32, (ROWS, K), 1)
    idx = jnp.min(jnp.where(dist == rowmin, kio, K), axis=1, keepdims=True)
    idx_ref[0, 0] = idx[:, 0]

    onehot = (kio == idx).astype(jnp.float32)
    zqt = jax.lax.dot_general(embt, onehot, (((1,), (1,)), ((), ())),
                              preferred_element_type=jnp.float32)
    zqt_ref[0] = zqt

    counts_acc[0:1, :] = counts_acc[0:1, :] + jnp.sum(onehot, axis=0,
                                                      keepdims=True)
    mind_acc[...] = mind_acc[...] + rowmin

    @pl.when(b == STEPS - 1)
    def _finish():
        counts = counts_acc[0:1, :]
        total = jnp.float32(N)
        probs = counts / total
        plogp = jnp.where(probs > 0.0, probs * jnp.log(
            jnp.where(probs > 0.0, probs, 1.0)), 0.0)
        h_ent = -jnp.sum(plogp)
        perplexity = jnp.exp(h_ent)
        codes_used = jnp.sum((counts > 0.0).astype(jnp.float32))
        avg_dist2 = jnp.sum(mind_acc[...]) / total
        loss_vq = (1.0 + BETA) * avg_dist2
        lane = jax.lax.broadcasted_iota(jnp.int32, (1, 8), 1)
        out = jnp.where(lane == 0, loss_vq,
              jnp.where(lane == 1, perplexity,
              jnp.where(lane == 2, codes_used,
              jnp.where(lane == 3, codes_used / jnp.float32(K),
              jnp.where(lane == 4, avg_dist2, 0.0)))))
        scal_ref[...] = out


@jax.jit
def _vq(z_rows, embt):
    grid = (STEPS,)
    idx_out, zqt_out, scal_out = pl.pallas_call(
        _vq_kernel,
        grid=grid,
        in_specs=[
            pl.BlockSpec((1, ROWS, D), lambda b: (b, 0, 0)),
            pl.BlockSpec((D, K), lambda b: (0, 0)),
        ],
        out_specs=[
            pl.BlockSpec((1, 1, ROWS), lambda b: (b, 0, 0)),
            pl.BlockSpec((1, D, ROWS), lambda b: (b, 0, 0)),
            pl.BlockSpec((1, 8), lambda b: (0, 0)),
        ],
        out_shape=[
            jax.ShapeDtypeStruct((STEPS, 1, ROWS), jnp.int32),
            jax.ShapeDtypeStruct((STEPS, D, ROWS), jnp.float32),
            jax.ShapeDtypeStruct((1, 8), jnp.float32),
        ],
        scratch_shapes=[
            pltpu.VMEM((8, K), jnp.float32),
            pltpu.VMEM((ROWS, 1), jnp.float32),
        ],
    )(z_rows.reshape(STEPS, ROWS, D), embt)
    return idx_out, zqt_out, scal_out


def kernel(z_e, emb):
    B, Dd, H, W = z_e.shape
    z_rows = jnp.transpose(z_e, (0, 2, 3, 1)).reshape(N, Dd)
    embt = emb.T
    idx_out, zqt_out, scal = _vq(z_rows, embt)
    indices = idx_out.reshape(B, H, W)
    z_q_st = zqt_out.reshape(B, Dd, H, W)
    loss_vq = scal[0, 0]
    perplexity = scal[0, 1]
    codes_used = scal[0, 2].astype(jnp.int32)
    usage_ratio = scal[0, 3]
    avg_dist2 = scal[0, 4]
    return (z_q_st, loss_vq, perplexity, codes_used, usage_ratio,
            avg_dist2, indices)
